# baseline (speedup 1.0000x reference)
import functools

import jax
import jax.numpy as jnp
from jax import lax
from jax.experimental import pallas as pl
from jax.experimental.pallas import tpu as pltpu
from jax.experimental.pallas import tpu_sc as plsc

EMBED_DIM = 256
NUM_POS_FEATS = 128
NUM_MASKS = 4
NUM_SLOTS = 7
L = 16
NW = 32
CHUNK = 16

S1 = 6.283088463027395
S3 = -41.333247542218885
S5 = 81.40008976706689
S7 = -74.67588386951014
S9 = 33.168094613349304


def _splat(x):
    return jnp.broadcast_to(x, (L,))


def _sin2pi(t):
    half = jnp.full((L,), 0.5, jnp.float32)
    one = jnp.full((L,), 1.0, jnp.float32)
    adj = jnp.where(t > half, one, jnp.zeros((L,), jnp.float32))
    adj = jnp.where(t < -half, -one, adj)
    t = t - adj
    z = t * t
    p = jnp.full((L,), S9, jnp.float32)
    p = p * z + jnp.full((L,), S7, jnp.float32)
    p = p * z + jnp.full((L,), S5, jnp.float32)
    p = p * z + jnp.full((L,), S3, jnp.float32)
    p = p * z + jnp.full((L,), S1, jnp.float32)
    return t * p


def _wrap(u):
    return u - lax.convert_element_type(
        lax.convert_element_type(u, jnp.int32), jnp.float32)


def kernel(points, feats_centers, pe_gaussian, corner_emb, point_emb, attr_W, mask_emb):
    B, Q, _ = points.shape
    R = B * Q
    rows_per_w = R // NW
    n_chunks = rows_per_w // CHUNK
    C = EMBED_DIM
    F = NUM_POS_FEATS
    JV = F // L

    pts_flat = points.reshape(R * 4)
    feats_flat = feats_centers.reshape(R, C)

    mesh = plsc.VectorSubcoreMesh(core_axis_name="c", subcore_axis_name="s",
                                  num_cores=2, num_subcores=16)

    @functools.partial(
        pl.kernel,
        mesh=mesh,
        out_type=jax.ShapeDtypeStruct((R, NUM_SLOTS, C), jnp.float32),
        scratch_types=[
            pltpu.VMEM((2, CHUNK, NUM_SLOTS, C), jnp.float32),
            pltpu.VMEM((2, CHUNK, C), jnp.float32),
            pltpu.VMEM((CHUNK * 4,), jnp.float32),
            pltpu.VMEM((2, F), jnp.float32),
            pltpu.VMEM((2, C), jnp.float32),
            pltpu.VMEM((NUM_MASKS + 1, C), jnp.float32),
            pltpu.VMEM((2, C), jnp.float32),
            pltpu.VMEM((1, C), jnp.float32),
            pltpu.VMEM((2, C), jnp.float32),
            pltpu.SemaphoreType.DMA,
            pltpu.SemaphoreType.DMA,
            pltpu.SemaphoreType.DMA,
            pltpu.SemaphoreType.DMA,
        ],
    )
    def sc_encode(pts_hbm, feats_hbm, pe_hbm, corner_hbm, point_hbm,
                  attr_hbm, mask_hbm, out_hbm,
                  buf, fv, pts_v, pe_v, bias_v, mask_v, corner_v, point_v,
                  attr_v, semo0, semo1, semi0, semi1):
        wid = lax.axis_index("s") * 2 + lax.axis_index("c")
        base = wid * rows_per_w

        pltpu.sync_copy(pe_hbm, pe_v)
        pltpu.sync_copy(corner_hbm.at[0], corner_v)
        pltpu.sync_copy(point_hbm.at[0], point_v)
        pltpu.sync_copy(attr_hbm, attr_v)
        pltpu.sync_copy(mask_hbm.at[0], mask_v)

        for k in range(2):
            for j in range(C // L):
                sl = pl.ds(j * L, L)
                bias_v[k, sl] = point_v[0, sl] + attr_v[1, sl] + corner_v[k, sl]

        def fill_template(i, _):
            for bi in range(2):
                for m in range(NUM_MASKS + 1):
                    for j in range(C // L):
                        sl = pl.ds(j * L, L)
                        buf[bi, i, 2 + m, sl] = mask_v[m, sl]
            return 0
        lax.fori_loop(0, CHUNK, fill_template, 0)

        def feats_copy(ci, sl_buf, semi):
            return pltpu.make_async_copy(
                feats_hbm.at[pl.ds(base + ci * CHUNK, CHUNK)],
                fv.at[sl_buf], semi)

        def pts_copy(ci):
            pltpu.sync_copy(
                pts_hbm.at[pl.ds((base + ci * CHUNK) * 4, CHUNK * 4)], pts_v)

        def out_copy(ci, sl_buf, semo):
            return pltpu.make_async_copy(
                buf.at[sl_buf],
                out_hbm.at[pl.ds(base + ci * CHUNK, CHUNK)], semo)

        feats_copy(0, 0, semi0).start()
        feats_copy(1, 1, semi1).start()

        inv = 2.0 / 1024.0

        def do_chunk(ci, bi, semi, semo):
            pts_copy(ci)
            feats_copy(ci, bi, semi).wait()

            @pl.when(ci >= 2)
            def _():
                out_copy(ci - 2, bi, semo).wait()

            def quad_body(qi, _):
                pv = pts_v[pl.ds(L * qi, L)]
                for r in range(4):
                    i = 4 * qi + r
                    for k in range(2):
                        cx = _splat(pv[4 * r + 2 * k]) * inv - 1.0
                        cy = _splat(pv[4 * r + 2 * k + 1]) * inv - 1.0
                        for j in range(JV):
                            sl = pl.ds(j * L, L)
                            sh = pl.ds(F + j * L, L)
                            u = cx * pe_v[0, sl] + cy * pe_v[1, sl]
                            ps = _sin2pi(_wrap(u))
                            pc = _sin2pi(_wrap(u + 0.25))
                            buf[bi, i, k, sl] = ps + bias_v[k, sl] + fv[bi, i, sl]
                            buf[bi, i, k, sh] = pc + bias_v[k, sh] + fv[bi, i, sh]
                return 0
            lax.fori_loop(0, CHUNK // 4, quad_body, 0)

            out_copy(ci, bi, semo).start()

            @pl.when(ci + 2 < n_chunks)
            def _():
                feats_copy(ci + 2, bi, semi).start()

        def pair_body(p, _):
            ci = 2 * p
            do_chunk(ci, 0, semi0, semo0)
            do_chunk(ci + 1, 1, semi1, semo1)
            return 0
        lax.fori_loop(0, n_chunks // 2, pair_body, 0)

        out_copy(n_chunks - 2, 0, semo0).wait()
        out_copy(n_chunks - 1, 1, semo1).wait()

    out = sc_encode(pts_flat, feats_flat, pe_gaussian, corner_emb,
                    point_emb, attr_W, mask_emb)
    out = out.reshape(B, Q, NUM_SLOTS, C)
    return (out, out)


# speedup vs baseline: 1.1173x; 1.1173x over previous
import functools

import jax
import jax.numpy as jnp
from jax import lax
from jax.experimental import pallas as pl
from jax.experimental.pallas import tpu as pltpu
from jax.experimental.pallas import tpu_sc as plsc

EMBED_DIM = 256
NUM_POS_FEATS = 128
NUM_MASKS = 4
NUM_SLOTS = 7
L = 16
NW = 32
CHUNK = 16

S1 = 6.283088463027395
S3 = -41.333247542218885
S5 = 81.40008976706689
S7 = -74.67588386951014
S9 = 33.168094613349304
C0 = 0.9999710807348359
C2 = -19.73279515561846
C4 = 64.71434198180282
C6 = -82.70097138611077
C8 = 46.30951922680341


def _splat(x):
    return jnp.broadcast_to(x, (L,))


def _sincos2pi(u):
    t = u - lax.convert_element_type(
        lax.convert_element_type(u, jnp.int32), jnp.float32)
    half = jnp.full((L,), 0.5, jnp.float32)
    one = jnp.full((L,), 1.0, jnp.float32)
    adj = jnp.where(t > half, one, jnp.zeros((L,), jnp.float32))
    adj = jnp.where(t < -half, -one, adj)
    t = t - adj
    z = t * t
    p = jnp.full((L,), S9, jnp.float32)
    p = p * z + jnp.full((L,), S7, jnp.float32)
    p = p * z + jnp.full((L,), S5, jnp.float32)
    p = p * z + jnp.full((L,), S3, jnp.float32)
    p = p * z + jnp.full((L,), S1, jnp.float32)
    q = jnp.full((L,), C8, jnp.float32)
    q = q * z + jnp.full((L,), C6, jnp.float32)
    q = q * z + jnp.full((L,), C4, jnp.float32)
    q = q * z + jnp.full((L,), C2, jnp.float32)
    q = q * z + jnp.full((L,), C0, jnp.float32)
    return t * p, q


def kernel(points, feats_centers, pe_gaussian, corner_emb, point_emb, attr_W, mask_emb):
    B, Q, _ = points.shape
    R = B * Q
    rows_per_w = R // NW
    n_chunks = rows_per_w // CHUNK
    C = EMBED_DIM
    F = NUM_POS_FEATS
    JV = F // L

    pts_flat = points.reshape(R * 4)
    feats_flat = feats_centers.reshape(R, C)

    mesh = plsc.VectorSubcoreMesh(core_axis_name="c", subcore_axis_name="s",
                                  num_cores=2, num_subcores=16)

    @functools.partial(
        pl.kernel,
        mesh=mesh,
        out_type=jax.ShapeDtypeStruct((R, NUM_SLOTS, C), jnp.float32),
        scratch_types=[
            pltpu.VMEM((2, CHUNK, NUM_SLOTS, C), jnp.float32),
            pltpu.VMEM((2, CHUNK, C), jnp.float32),
            pltpu.VMEM((CHUNK * 4,), jnp.float32),
            pltpu.VMEM((2, F), jnp.float32),
            pltpu.VMEM((2, C), jnp.float32),
            pltpu.VMEM((NUM_MASKS + 1, C), jnp.float32),
            pltpu.VMEM((2, C), jnp.float32),
            pltpu.VMEM((1, C), jnp.float32),
            pltpu.VMEM((2, C), jnp.float32),
            pltpu.SemaphoreType.DMA,
            pltpu.SemaphoreType.DMA,
            pltpu.SemaphoreType.DMA,
            pltpu.SemaphoreType.DMA,
        ],
    )
    def sc_encode(pts_hbm, feats_hbm, pe_hbm, corner_hbm, point_hbm,
                  attr_hbm, mask_hbm, out_hbm,
                  buf, fv, pts_v, pe_v, bias_v, mask_v, corner_v, point_v,
                  attr_v, semo0, semo1, semi0, semi1):
        wid = lax.axis_index("s") * 2 + lax.axis_index("c")
        base = wid * rows_per_w

        pltpu.sync_copy(pe_hbm, pe_v)
        pltpu.sync_copy(corner_hbm.at[0], corner_v)
        pltpu.sync_copy(point_hbm.at[0], point_v)
        pltpu.sync_copy(attr_hbm, attr_v)
        pltpu.sync_copy(mask_hbm.at[0], mask_v)

        for k in range(2):
            for j in range(C // L):
                sl = pl.ds(j * L, L)
                bias_v[k, sl] = point_v[0, sl] + attr_v[1, sl] + corner_v[k, sl]

        def fill_template(i, _):
            for bi in range(2):
                for m in range(NUM_MASKS + 1):
                    for j in range(C // L):
                        sl = pl.ds(j * L, L)
                        buf[bi, i, 2 + m, sl] = mask_v[m, sl]
            return 0
        lax.fori_loop(0, CHUNK, fill_template, 0)

        def feats_copy(ci, sl_buf, semi):
            return pltpu.make_async_copy(
                feats_hbm.at[pl.ds(base + ci * CHUNK, CHUNK)],
                fv.at[sl_buf], semi)

        def pts_copy(ci):
            pltpu.sync_copy(
                pts_hbm.at[pl.ds((base + ci * CHUNK) * 4, CHUNK * 4)], pts_v)

        def out_copy(ci, sl_buf, semo):
            return pltpu.make_async_copy(
                buf.at[sl_buf],
                out_hbm.at[pl.ds(base + ci * CHUNK, CHUNK)], semo)

        feats_copy(0, 0, semi0).start()
        feats_copy(1, 1, semi1).start()

        inv = 2.0 / 1024.0

        def do_chunk(ci, bi, semi, semo):
            pts_copy(ci)
            feats_copy(ci, bi, semi).wait()

            @pl.when(ci >= 2)
            def _():
                out_copy(ci - 2, bi, semo).wait()

            def quad_body(qi, _):
                pv = pts_v[pl.ds(L * qi, L)]
                for r in range(4):
                    i = 4 * qi + r
                    for k in range(2):
                        cx = _splat(pv[4 * r + 2 * k]) * inv - 1.0
                        cy = _splat(pv[4 * r + 2 * k + 1]) * inv - 1.0
                        for j in range(JV):
                            sl = pl.ds(j * L, L)
                            sh = pl.ds(F + j * L, L)
                            u = cx * pe_v[0, sl] + cy * pe_v[1, sl]
                            ps, pc = _sincos2pi(u)
                            buf[bi, i, k, sl] = ps + bias_v[k, sl] + fv[bi, i, sl]
                            buf[bi, i, k, sh] = pc + bias_v[k, sh] + fv[bi, i, sh]
                return 0
            lax.fori_loop(0, CHUNK // 4, quad_body, 0)

            out_copy(ci, bi, semo).start()

            @pl.when(ci + 2 < n_chunks)
            def _():
                feats_copy(ci + 2, bi, semi).start()

        def pair_body(p, _):
            ci = 2 * p
            do_chunk(ci, 0, semi0, semo0)
            do_chunk(ci + 1, 1, semi1, semo1)
            return 0
        lax.fori_loop(0, n_chunks // 2, pair_body, 0)

        out_copy(n_chunks - 2, 0, semo0).wait()
        out_copy(n_chunks - 1, 1, semo1).wait()

    out = sc_encode(pts_flat, feats_flat, pe_gaussian, corner_emb,
                    point_emb, attr_W, mask_emb)
    out = out.reshape(B, Q, NUM_SLOTS, C)
    return (out, out)


# speedup vs baseline: 3.5684x; 3.1939x over previous
import math

import jax
import jax.numpy as jnp
from jax.experimental import pallas as pl

EMBED_DIM = 256
NUM_POS_FEATS = EMBED_DIM // 2
IMAGE_SIZE = (1024, 1024)
NUM_MASKS = 4


def _encoder_body(points_ref, feats_ref, pe_ref, corner_ref, point_ref,
                  attr_ref, mask_ref, out_ref):
    pts = points_ref[0]
    feats = feats_ref[0]
    g0 = pe_ref[0]
    g1 = pe_ref[1]
    base = point_ref[0, 0] + attr_ref[1]

    two_pi = 2.0 * math.pi
    sx = two_pi * (2.0 / IMAGE_SIZE[1])
    sy = two_pi * (2.0 / IMAGE_SIZE[0])

    q = pts.shape[0]
    for k in range(2):
        x = pts[:, 2 * k] * sx - two_pi
        y = pts[:, 2 * k + 1] * sy - two_pi
        arg = x[:, None] * g0[None, :] + y[:, None] * g1[None, :]
        pe = jnp.concatenate([jnp.sin(arg), jnp.cos(arg)], axis=-1)
        out_ref[0, :, k, :] = pe + (base + corner_ref[0, k])[None, :] + feats
    out_ref[0, :, 2:, :] = jnp.broadcast_to(mask_ref[0][None], (q, NUM_MASKS + 1, EMBED_DIM))


def kernel(points, feats_centers, pe_gaussian, corner_emb, point_emb, attr_W, mask_emb):
    B, Q, _ = points.shape
    C = EMBED_DIM
    S = 2 + NUM_MASKS + 1
    out = pl.pallas_call(
        _encoder_body,
        grid=(B, 2),
        in_specs=[
            pl.BlockSpec((1, Q // 2, 4), lambda b, qq: (b, qq, 0)),
            pl.BlockSpec((1, Q // 2, C), lambda b, qq: (b, qq, 0)),
            pl.BlockSpec((2, NUM_POS_FEATS), lambda b, qq: (0, 0)),
            pl.BlockSpec((1, 2, C), lambda b, qq: (0, 0, 0)),
            pl.BlockSpec((1, 1, C), lambda b, qq: (0, 0, 0)),
            pl.BlockSpec((2, C), lambda b, qq: (0, 0)),
            pl.BlockSpec((1, NUM_MASKS + 1, C), lambda b, qq: (0, 0, 0)),
        ],
        out_specs=pl.BlockSpec((1, Q // 2, S, C), lambda b, qq: (b, qq, 0, 0)),
        out_shape=jax.ShapeDtypeStruct((B, Q, S, C), jnp.float32),
    )(points, feats_centers, pe_gaussian, corner_emb, point_emb, attr_W, mask_emb)
    return (out, out)


# speedup vs baseline: 3.7264x; 1.0443x over previous
import math

import jax
import jax.numpy as jnp
from jax.experimental import pallas as pl

EMBED_DIM = 256
NUM_POS_FEATS = EMBED_DIM // 2
IMAGE_SIZE = (1024, 1024)
NUM_MASKS = 4


def _encoder_body(points_ref, feats_ref, pe_ref, corner_ref, point_ref,
                  attr_ref, mask_ref, out_ref):
    pts = points_ref[...].reshape(-1, 4)
    feats = feats_ref[...].reshape(-1, EMBED_DIM)
    g0 = pe_ref[0]
    g1 = pe_ref[1]
    base = point_ref[0, 0] + attr_ref[1]

    two_pi = 2.0 * math.pi
    sx = two_pi * (2.0 / IMAGE_SIZE[1])
    sy = two_pi * (2.0 / IMAGE_SIZE[0])

    q = pts.shape[0]
    for k in range(2):
        x = pts[:, 2 * k] * sx - two_pi
        y = pts[:, 2 * k + 1] * sy - two_pi
        arg = x[:, None] * g0[None, :] + y[:, None] * g1[None, :]
        pe = jnp.concatenate([jnp.sin(arg), jnp.cos(arg)], axis=-1)
        v = pe + (base + corner_ref[0, k])[None, :] + feats
        out_ref[0, :, k, :] = v[:q // 2]
        out_ref[1, :, k, :] = v[q // 2:]
    bc = jnp.broadcast_to(mask_ref[0][None], (q // 2, NUM_MASKS + 1, EMBED_DIM))
    out_ref[0, :, 2:, :] = bc
    out_ref[1, :, 2:, :] = bc


def kernel(points, feats_centers, pe_gaussian, corner_emb, point_emb, attr_W, mask_emb):
    B, Q, _ = points.shape
    C = EMBED_DIM
    S = 2 + NUM_MASKS + 1
    out = pl.pallas_call(
        _encoder_body,
        grid=(B // 2,),
        in_specs=[
            pl.BlockSpec((2, Q, 4), lambda b: (b, 0, 0)),
            pl.BlockSpec((2, Q, C), lambda b: (b, 0, 0)),
            pl.BlockSpec((2, NUM_POS_FEATS), lambda b: (0, 0)),
            pl.BlockSpec((1, 2, C), lambda b: (0, 0, 0)),
            pl.BlockSpec((1, 1, C), lambda b: (0, 0, 0)),
            pl.BlockSpec((2, C), lambda b: (0, 0)),
            pl.BlockSpec((1, NUM_MASKS + 1, C), lambda b: (0, 0, 0)),
        ],
        out_specs=pl.BlockSpec((2, Q, S, C), lambda b: (b, 0, 0, 0)),
        out_shape=jax.ShapeDtypeStruct((B, Q, S, C), jnp.float32),
    )(points, feats_centers, pe_gaussian, corner_emb, point_emb, attr_W, mask_emb)
    return (out, out)
